# baseline (speedup 1.0000x reference)
import jax
import jax.numpy as jnp
from jax import lax
from jax.experimental import pallas as pl
from jax.experimental.pallas import tpu as pltpu
from jax.experimental.pallas import tpu_sc as plsc

_LN2 = 0.6931471805599453
_BATCH = 16384
_NC = 2
_NS = 16
_LANES = 16
_NW = _NC * _NS
_B_PER_W = _BATCH // _NW
_HALF = _B_PER_W // 2


def _body(ids_hbm, vars_hbm, out_hbm, idx_v, rows_v, sem, gsem0, gsem1, wsem):
    gsems = (gsem0, gsem1)
    wid = lax.axis_index("s") * _NC + lax.axis_index("c")
    base = wid * _B_PER_W
    for h in range(2):
        pltpu.async_copy(
            ids_hbm.at[pl.ds(base + h * _HALF, _HALF)],
            idx_v.at[pl.ds(h * _HALF, _HALF)], gsems[h])
    for h in range(2):
        pltpu.make_async_copy(
            ids_hbm.at[pl.ds(base + h * _HALF, _HALF)],
            idx_v.at[pl.ds(h * _HALF, _HALF)], gsems[h]).wait()
        pltpu.async_copy(
            vars_hbm.at[idx_v.at[pl.ds(h * _HALF, _HALF)]],
            rows_v.at[pl.ds(h * _HALF, _HALF)], gsems[h])
    for h in range(2):
        pltpu.make_async_copy(
            vars_hbm.at[idx_v.at[pl.ds(h * _HALF, _HALF)]],
            rows_v.at[pl.ds(h * _HALF, _HALF)], gsems[h]).wait()

        for i in range(_HALF // _LANES):
            v = rows_v[pl.ds(h * _HALF + i * _LANES, _LANES)]
            rows_v[pl.ds(h * _HALF + i * _LANES, _LANES)] = jnp.exp(v * _LN2)
        pltpu.async_copy(
            rows_v.at[pl.ds(h * _HALF, _HALF)],
            out_hbm.at[pl.ds(base + h * _HALF, _HALF)], wsem)
    for h in range(2):
        pltpu.make_async_copy(
            rows_v.at[pl.ds(h * _HALF, _HALF)],
            out_hbm.at[pl.ds(base + h * _HALF, _HALF)], wsem).wait()


@jax.jit
def kernel(ids, vars_):
    mesh = plsc.VectorSubcoreMesh(core_axis_name="c", subcore_axis_name="s")
    run = pl.kernel(
        _body,
        out_type=jax.ShapeDtypeStruct((_BATCH,), jnp.float32),
        mesh=mesh,
        scratch_types=[
            pltpu.VMEM((_B_PER_W,), jnp.int32),
            pltpu.VMEM((_B_PER_W,), jnp.float32),
            pltpu.SemaphoreType.DMA,
            pltpu.SemaphoreType.DMA,
            pltpu.SemaphoreType.DMA,
            pltpu.SemaphoreType.DMA,
        ],
    )
    return run(ids.astype(jnp.int32), vars_)


# speedup vs baseline: 1.0045x; 1.0045x over previous
import jax
import jax.numpy as jnp
from jax import lax
from jax.experimental import pallas as pl
from jax.experimental.pallas import tpu as pltpu
from jax.experimental.pallas import tpu_sc as plsc

_LN2 = 0.6931471805599453
_BATCH = 16384
_NC = 2
_NS = 16
_LANES = 16
_NW = _NC * _NS
_B_PER_W = _BATCH // _NW
_HALF = _B_PER_W // 2


def _body(ids_hbm, vars_hbm, out_hbm, idx_v, rows_v, sem, gsem0, gsem1, wsem):
    gsems = (gsem0, gsem1)
    wid = lax.axis_index("s") * _NC + lax.axis_index("c")
    base = wid * _B_PER_W
    ids_src = ids_hbm.at[pl.ds(base, _B_PER_W)]
    pltpu.async_copy(ids_src, idx_v, sem)
    pltpu.make_async_copy(ids_src, idx_v, sem).wait()
    for h in range(2):
        pltpu.async_copy(
            vars_hbm.at[idx_v.at[pl.ds(h * _HALF, _HALF)]],
            rows_v.at[pl.ds(h * _HALF, _HALF)], gsems[h])
    for h in range(2):
        pltpu.make_async_copy(
            vars_hbm.at[idx_v.at[pl.ds(h * _HALF, _HALF)]],
            rows_v.at[pl.ds(h * _HALF, _HALF)], gsems[h]).wait()

        for i in range(_HALF // _LANES):
            v = rows_v[pl.ds(h * _HALF + i * _LANES, _LANES)]
            rows_v[pl.ds(h * _HALF + i * _LANES, _LANES)] = jnp.exp(v * _LN2)
        pltpu.async_copy(
            rows_v.at[pl.ds(h * _HALF, _HALF)],
            out_hbm.at[pl.ds(base + h * _HALF, _HALF)], wsem)
    for h in range(2):
        pltpu.make_async_copy(
            rows_v.at[pl.ds(h * _HALF, _HALF)],
            out_hbm.at[pl.ds(base + h * _HALF, _HALF)], wsem).wait()


@jax.jit
def kernel(ids, vars_):
    mesh = plsc.VectorSubcoreMesh(core_axis_name="c", subcore_axis_name="s")
    run = pl.kernel(
        _body,
        out_type=jax.ShapeDtypeStruct((_BATCH,), jnp.float32),
        mesh=mesh,
        scratch_types=[
            pltpu.VMEM((_B_PER_W,), jnp.int32),
            pltpu.VMEM((_B_PER_W,), jnp.float32),
            pltpu.SemaphoreType.DMA,
            pltpu.SemaphoreType.DMA,
            pltpu.SemaphoreType.DMA,
            pltpu.SemaphoreType.DMA,
        ],
    )
    return run(ids.astype(jnp.int32), vars_)
